# Initial kernel scaffold; baseline (speedup 1.0000x reference)
import jax
import jax.numpy as jnp
from jax.experimental import pallas as pl


def kernel(x_node, rw_pos_enc, edge_attr, subgraphs_nodes_mapper, subgraphs_edges_mapper, combined_subgraphs, subgraphs_batch, patch_pe, coarsen_adj, mask, atom_emb, bond_emb, rw_W, rw_b, patch_W, patch_b, gnn_W, gnn_b, U_W, U_b, Wq, Wk, Wv, Wo, ffW1, ffb1, ffW2, ffb2, oW1, ob1, oW2, ob2):
    raise NotImplementedError("write your pallas kernel here")


# speedup vs baseline: 1.0002x; 1.0002x over previous
import functools

import jax
import jax.numpy as jnp
from jax.experimental import pallas as pl
from jax.experimental.pallas import tpu as pltpu

N = 10000; E = 320000; NS = 40000; ES = 400000
B = 8; P = 32; TP = B * P; H = 128
NL_GNN = 4; NL_MIX = 2


def _seg_mean(data, ids, n):
    s = jax.ops.segment_sum(data, ids, num_segments=n)
    c = jax.ops.segment_sum(jnp.ones((ids.shape[0],), dtype=data.dtype), ids,
                            num_segments=n)
    return s / jnp.clip(c, 1.0)[:, None]


def _ln(x):
    m = x.mean(-1, keepdims=True)
    v = x.var(-1, keepdims=True)
    return (x - m) * jax.lax.rsqrt(v + 1e-5)


def _mixer_body(sx_ref, adj_ref, maskf_ref, wq_ref, wk_ref, wv_ref, wo_ref,
                w1_ref, b1_ref, w2_ref, b2_ref, ow1_ref, ob1_ref, ow2_ref,
                ob2_ref, out_ref):
    mx = sx_ref[...]
    maskf = maskf_ref[...]
    scale = 1.0 / (H ** 0.5)
    for l in range(NL_MIX):
        h = _ln(mx)
        new_rows = []
        for b in range(B):
            hb = h[b * P:(b + 1) * P, :]
            qb = jnp.dot(hb, wq_ref[l], preferred_element_type=jnp.float32)
            kb = jnp.dot(hb, wk_ref[l], preferred_element_type=jnp.float32)
            vb = jnp.dot(hb, wv_ref[l], preferred_element_type=jnp.float32)
            sc = jnp.dot(qb, kb.T, preferred_element_type=jnp.float32) * scale
            sc = jnp.where((maskf[b] == 0.0)[None, :], -1e9, sc)
            sc = sc - jnp.max(sc, axis=-1, keepdims=True)
            e = jnp.exp(sc)
            att = e / jnp.sum(e, axis=-1, keepdims=True)
            att = att * adj_ref[b]
            ob = jnp.dot(jnp.dot(att, vb, preferred_element_type=jnp.float32),
                         wo_ref[l], preferred_element_type=jnp.float32)
            new_rows.append(ob)
        mx = mx + jnp.concatenate(new_rows, axis=0)
        h2 = _ln(mx)
        ff = jnp.maximum(jnp.dot(h2, w1_ref[l],
                                 preferred_element_type=jnp.float32) + b1_ref[l], 0.0)
        mx = mx + jnp.dot(ff, w2_ref[l],
                          preferred_element_type=jnp.float32) + b2_ref[l]
    mx3 = mx.reshape(B, P, H)
    wsum = jnp.sum(mx3 * maskf[:, :, None], axis=1)
    cnt = jnp.sum(maskf, axis=1, keepdims=True)
    g = wsum / cnt
    dec = jnp.maximum(jnp.dot(g, ow1_ref[...],
                              preferred_element_type=jnp.float32) + ob1_ref[...], 0.0)
    out_ref[...] = jnp.dot(dec, ow2_ref[...],
                           preferred_element_type=jnp.float32) + ob2_ref[...]


def _mixer(sx, coarsen_adj, maskf, Wq, Wk, Wv, Wo, ffW1, ffb1, ffW2, ffb2,
           oW1, ob1, oW2, ob2):
    return pl.pallas_call(
        _mixer_body,
        out_shape=jax.ShapeDtypeStruct((B, 1), jnp.float32),
    )(sx, coarsen_adj, maskf, Wq, Wk, Wv, Wo, ffW1, ffb1, ffW2, ffb2,
      oW1, ob1, oW2, ob2)


def kernel(x_node, rw_pos_enc, edge_attr, subgraphs_nodes_mapper,
           subgraphs_edges_mapper, combined_subgraphs, subgraphs_batch,
           patch_pe, coarsen_adj, mask, atom_emb, bond_emb, rw_W, rw_b,
           patch_W, patch_b, gnn_W, gnn_b, U_W, U_b, Wq, Wk, Wv, Wo,
           ffW1, ffb1, ffW2, ffb2, oW1, ob1, oW2, ob2):
    x = atom_emb[x_node]
    x = x + rw_pos_enc @ rw_W + rw_b
    ea = bond_emb[edge_attr]
    x = x[subgraphs_nodes_mapper]
    e = ea[subgraphs_edges_mapper]
    src = combined_subgraphs[0]
    dst = combined_subgraphs[1]
    for i in range(NL_GNN):
        if i > 0:
            sub = _seg_mean(x, subgraphs_batch, TP)[subgraphs_batch]
            x = x + jax.nn.relu(sub @ U_W[i - 1] + U_b[i - 1])
            x = _seg_mean(x, subgraphs_nodes_mapper, N)[subgraphs_nodes_mapper]
        m = jax.nn.relu(x[src] + e)
        agg = jax.ops.segment_sum(m, dst, num_segments=NS)
        h = jax.nn.relu((x + agg) @ gnn_W[i] + gnn_b[i])
        x = x + h
    sx = _seg_mean(x, subgraphs_batch, TP)
    sx = sx + patch_pe @ patch_W + patch_b
    maskf = mask.astype(jnp.float32)
    return _mixer(sx, coarsen_adj, maskf, Wq, Wk, Wv, Wo, ffW1, ffb1,
                  ffW2, ffb2, oW1, ob1, oW2, ob2)


# speedup vs baseline: 1.8847x; 1.8843x over previous
import functools

import jax
import jax.numpy as jnp
from jax import lax
from jax.experimental import pallas as pl
from jax.experimental.pallas import tpu as pltpu
from jax.experimental.pallas import tpu_sc as plsc

N = 10000; E = 320000; NS = 40000; ES = 400000
B = 8; P = 32; TP = B * P; H = 128
NL_GNN = 4; NL_MIX = 2

NP_ = 10240
NSP = 40960
ESP = 409600
SB = 272
TBL = 10 * NP_
NT = 16
EPT = ESP // NT
ECH = EPT // 128
RPT = NSP // NT
RCH = RPT // 128
NPT = NP_ // NT
EW = ESP // (2 * NT)
EWC = EW // 128

_f32 = jnp.float32
_i32 = jnp.int32


def _sds(shape, dtype=_f32):
    return jax.ShapeDtypeStruct(shape, dtype)


def _al(x):
    return pl.multiple_of(x, 8)


def _prep_body(sem2, src2, ea_t, snm_t, batchS2, snm2, ones_in, zrows,
               enc_all, eidx_out, cntn_out, cntb_out, x0_out,
               cS, b1, b2, jb, kb, av, nv, rows, ov, gs0, gs1):
    c = lax.axis_index("c")
    s = lax.axis_index("s")
    w = c * NT + s
    pltpu.sync_copy(sem2.at[w], b1)
    pltpu.sync_copy(src2.at[w], b2)

    def idx_chunk(k, _):
        pltpu.async_copy(ea_t.at[b1.at[k]], av, gs0).wait()
        pltpu.async_copy(snm_t.at[b2.at[k]], nv, gs1).wait()
        for i in range(8):
            sl = pl.ds(i * 16, 16)
            b1[k, sl] = av[sl] * NP_ + nv[sl]
        return 0

    lax.fori_loop(0, EWC, idx_chunk, 0)
    pltpu.sync_copy(b1, eidx_out.at[w])

    pltpu.sync_copy(snm2.at[s], jb)
    pltpu.sync_copy(batchS2.at[s], kb)
    pltpu.sync_copy(ones_in, ov)
    ZPT = (NP_ + 512) // NT
    pltpu.sync_copy(zrows.at[pl.ds(_al(s * ZPT), ZPT)],
                    cS.at[pl.ds(_al(s * ZPT), ZPT)])
    plsc.subcore_barrier()

    def cnt_chunk(k, _):
        pltpu.sync_copy(ov, cS.at[jb.at[k]], add=True)
        pltpu.sync_copy(ov, cS.at[kb.at[k]], add=True)
        return 0

    lax.fori_loop(0, RCH, cnt_chunk, 0)
    plsc.subcore_barrier()

    @pl.when(c == 0)
    def _():
        pltpu.sync_copy(cS.at[pl.ds(_al(s * NPT), NPT)],
                        cntn_out.at[pl.ds(_al(s * NPT), NPT)])

    @pl.when(jnp.logical_and(c == 1, s == 0))
    def _():
        pltpu.sync_copy(cS.at[pl.ds(NP_, SB)], cntb_out)

    for j in range(2):
        q = c * 2 + j

        def g_chunk(k, _):
            base = s * RPT + k * 128
            for i in range(8):
                sl = pl.ds(i * 16, 16)
                kb[k, sl] = jb[k, sl] + q * NP_
            pltpu.async_copy(enc_all.at[kb.at[k]], rows, gs0).wait()
            pltpu.sync_copy(rows, x0_out.at[pl.ds(_al(q * NSP + base), 128)])
            return 0

        lax.fori_loop(0, RCH, g_chunk, 0)


def _edge_body(tb_all, eidx2, dst2, zrows, agg_out,
               aggS, eb, db, eb2, rows, gs0, gs1):
    c = lax.axis_index("c")
    s = lax.axis_index("s")
    CPB = 40
    NBLK = ECH // CPB
    for j in range(2):
        q = c * 2 + j
        pltpu.sync_copy(zrows.at[pl.ds(_al(s * RPT), RPT)],
                        aggS.at[pl.ds(_al(s * RPT), RPT)])
        plsc.subcore_barrier()

        def block(bk, _):
            base = _al(s * ECH + bk * CPB)
            pltpu.sync_copy(eidx2.at[pl.ds(base, CPB)], eb)
            pltpu.sync_copy(dst2.at[pl.ds(base, CPB)], db)

            def shift_chunk(k, _):
                for i in range(8):
                    sl = pl.ds(i * 16, 16)
                    eb2[k, sl] = eb[k, sl] + q * TBL
                return 0

            lax.fori_loop(0, CPB, shift_chunk, 0)
            pltpu.async_copy(tb_all.at[eb2.at[0]], rows.at[0], gs0)

            def pair(p, _):
                c0 = 2 * p
                pltpu.make_async_copy(tb_all.at[eb2.at[c0]], rows.at[0],
                                      gs0).wait()
                pltpu.async_copy(tb_all.at[eb2.at[c0 + 1]], rows.at[1], gs1)
                pltpu.sync_copy(rows.at[0], aggS.at[db.at[c0]], add=True)
                pltpu.make_async_copy(tb_all.at[eb2.at[c0 + 1]], rows.at[1],
                                      gs1).wait()

                @pl.when(p < CPB // 2 - 1)
                def _():
                    pltpu.async_copy(tb_all.at[eb2.at[c0 + 2]], rows.at[0],
                                     gs0)

                pltpu.sync_copy(rows.at[1], aggS.at[db.at[c0 + 1]], add=True)
                return 0

            lax.fori_loop(0, CPB // 2, pair, 0)
            return 0

        lax.fori_loop(0, NBLK, block, 0)
        plsc.subcore_barrier()
        pltpu.sync_copy(aggS.at[pl.ds(_al(s * RPT), RPT)],
                        agg_out.at[pl.ds(_al(q * NSP + s * RPT), RPT)])
        plsc.subcore_barrier()


def _bsum_body(x_all, batch2, zrows, sb_out, sS, ib, rows, gs0, gs1):
    c = lax.axis_index("c")
    s = lax.axis_index("s")
    pltpu.sync_copy(batch2.at[s], ib)
    for j in range(2):
        q = c * 2 + j

        @pl.when(s == 0)
        def _():
            pltpu.sync_copy(zrows.at[pl.ds(0, SB)], sS)

        plsc.subcore_barrier()

        pltpu.async_copy(x_all.at[pl.ds(_al(q * NSP + s * RPT), 128)],
                         rows.at[0], gs0)

        def pair(p, _):
            base = q * NSP + s * RPT
            c0 = 2 * p
            pltpu.make_async_copy(
                x_all.at[pl.ds(_al(base + c0 * 128), 128)], rows.at[0],
                gs0).wait()
            pltpu.async_copy(x_all.at[pl.ds(_al(base + (c0 + 1) * 128), 128)],
                             rows.at[1], gs1)
            pltpu.sync_copy(rows.at[0], sS.at[ib.at[c0]], add=True)
            pltpu.make_async_copy(
                x_all.at[pl.ds(_al(base + (c0 + 1) * 128), 128)], rows.at[1],
                gs1).wait()

            @pl.when(p < RCH // 2 - 1)
            def _():
                pltpu.async_copy(
                    x_all.at[pl.ds(_al(base + (c0 + 2) * 128), 128)],
                    rows.at[0], gs0)

            pltpu.sync_copy(rows.at[1], sS.at[ib.at[c0 + 1]], add=True)
            return 0

        lax.fori_loop(0, RCH // 2, pair, 0)
        plsc.subcore_barrier()
        pltpu.sync_copy(sS.at[pl.ds(_al(s * 16), 16)],
                        sb_out.at[pl.ds(_al(q * SB + s * 16), 16)])

        @pl.when(s == 0)
        def _():
            pltpu.sync_copy(sS.at[pl.ds(256, 16)],
                            sb_out.at[pl.ds(_al(q * SB + 256), 16)])

        plsc.subcore_barrier()


def _node_body(x_all, R_all, batch2, snm2, cntn, zrows, node_out, xn_out,
               nS, ib, jb, jb2, rows, rrows, sbuf, tbuf, gs0, gs1):
    c = lax.axis_index("c")
    s = lax.axis_index("s")
    pltpu.sync_copy(batch2.at[s], ib)
    pltpu.sync_copy(snm2.at[s], jb)
    for j in range(2):
        q = c * 2 + j
        pltpu.sync_copy(zrows.at[pl.ds(_al(s * NPT), NPT)],
                        nS.at[pl.ds(_al(s * NPT), NPT)])

        def shift_chunk(k, _):
            for i in range(8):
                sl = pl.ds(i * 16, 16)
                jb2[k, sl] = ib[k, sl] + q * SB
            return 0

        lax.fori_loop(0, RCH, shift_chunk, 0)
        plsc.subcore_barrier()

        def acc_chunk(k, _):
            base = _al(q * NSP + s * RPT + k * 128)
            pltpu.async_copy(x_all.at[pl.ds(base, 128)], rows, gs0)
            pltpu.async_copy(R_all.at[jb2.at[k]], rrows, gs1)
            pltpu.make_async_copy(x_all.at[pl.ds(base, 128)], rows, gs0).wait()
            pltpu.make_async_copy(R_all.at[jb2.at[k]], rrows, gs1).wait()

            def addrow(r, _):
                for hh in range(2):
                    sl = pl.ds(hh * 16, 16)
                    rows[r, sl] = rows[r, sl] + rrows[r, sl]
                return 0

            lax.fori_loop(0, 128, addrow, 0)
            pltpu.sync_copy(rows, nS.at[jb.at[k]], add=True)
            return 0

        lax.fori_loop(0, RCH, acc_chunk, 0)
        plsc.subcore_barrier()
        pltpu.sync_copy(nS.at[pl.ds(_al(s * NPT), NPT)], sbuf)
        pltpu.sync_copy(cntn.at[pl.ds(_al(s * NPT), NPT)], tbuf)

        def scale_row(r, _):
            for hh in range(2):
                sl = pl.ds(hh * 16, 16)
                sbuf[r, sl] = sbuf[r, sl] / jnp.maximum(tbuf[r, sl], 1.0)
            return 0

        lax.fori_loop(0, NPT, scale_row, 0)
        pltpu.sync_copy(sbuf,
                        node_out.at[pl.ds(_al(q * NP_ + s * NPT), NPT)])
        plsc.subcore_barrier()

        def gb_chunk(k, _):
            for i in range(8):
                sl = pl.ds(i * 16, 16)
                jb2[k, sl] = jb[k, sl] + q * NP_
            base = _al(q * NSP + s * RPT + k * 128)
            pltpu.async_copy(node_out.at[jb2.at[k]], rows, gs0).wait()
            pltpu.sync_copy(rows, xn_out.at[pl.ds(base, 128)])
            return 0

        lax.fori_loop(0, RCH, gb_chunk, 0)


def _mesh():
    return plsc.VectorSubcoreMesh(core_axis_name="c", subcore_axis_name="s",
                                  num_cores=2, num_subcores=NT)


def _prep_sc(body):
  return pl.kernel(
    body,
    compiler_params=pltpu.CompilerParams(use_tc_tiling_on_sc=False),
    out_type=(_sds((2 * NT, EWC, 128), _i32), _sds((NP_, 32)),
              _sds((SB, 32)), _sds((4 * NSP, 32))),
    mesh=_mesh(),
    scratch_types=[
        pltpu.VMEM_SHARED((NP_ + 512, 32), _f32),
        pltpu.VMEM((EWC, 128), _i32),
        pltpu.VMEM((EWC, 128), _i32),
        pltpu.VMEM((RCH, 128), _i32),
        pltpu.VMEM((RCH, 128), _i32),
        pltpu.VMEM((128,), _i32),
        pltpu.VMEM((128,), _i32),
        pltpu.VMEM((128, 32), _f32),
        pltpu.VMEM((128, 32), _f32),
        pltpu.SemaphoreType.DMA,
        pltpu.SemaphoreType.DMA,
    ])

def _edge_sc(body):
  return pl.kernel(
    body,
    compiler_params=pltpu.CompilerParams(use_tc_tiling_on_sc=False),
    out_type=_sds((4 * NSP, 32)),
    mesh=_mesh(),
    scratch_types=[
        pltpu.VMEM_SHARED((NSP, 32), _f32),
        pltpu.VMEM((40, 128), _i32),
        pltpu.VMEM((40, 128), _i32),
        pltpu.VMEM((40, 128), _i32),
        pltpu.VMEM((2, 128, 32), _f32),
        pltpu.SemaphoreType.DMA,
        pltpu.SemaphoreType.DMA,
    ])

def _bsum_sc(body):
  return pl.kernel(
    body,
    compiler_params=pltpu.CompilerParams(use_tc_tiling_on_sc=False),
    out_type=_sds((4 * SB, 32)),
    mesh=_mesh(),
    scratch_types=[
        pltpu.VMEM_SHARED((SB, 32), _f32),
        pltpu.VMEM((RCH, 128), _i32),
        pltpu.VMEM((2, 128, 32), _f32),
        pltpu.SemaphoreType.DMA,
        pltpu.SemaphoreType.DMA,
    ])

def _node_sc(body):
  return pl.kernel(
    body,
    compiler_params=pltpu.CompilerParams(use_tc_tiling_on_sc=False),
    out_type=(_sds((4 * NP_, 32)), _sds((4 * NSP, 32))),
    mesh=_mesh(),
    scratch_types=[
        pltpu.VMEM_SHARED((NP_, 32), _f32),
        pltpu.VMEM((RCH, 128), _i32),
        pltpu.VMEM((RCH, 128), _i32),
        pltpu.VMEM((RCH, 128), _i32),
        pltpu.VMEM((128, 32), _f32),
        pltpu.VMEM((128, 32), _f32),
        pltpu.VMEM((NPT, 32), _f32),
        pltpu.VMEM((NPT, 32), _f32),
        pltpu.SemaphoreType.DMA,
        pltpu.SemaphoreType.DMA,
    ])


def _enc_body(xn_ref, rw_ref, atom_ref, rww_ref, rwb_ref, out_ref):
    xn = xn_ref[...]
    oh = (xn == lax.broadcasted_iota(_i32, (1024, 128), 1)).astype(_f32)
    enc = jnp.dot(oh, atom_ref[...], preferred_element_type=_f32)
    enc = enc + jnp.dot(rw_ref[...], rww_ref[...],
                        preferred_element_type=_f32) + rwb_ref[...]
    out_ref[...] = jnp.concatenate(
        [enc[:, 32 * q:32 * (q + 1)].reshape(1, 1024, 32) for q in range(4)],
        axis=0)


def _enc_tc(xnp, rwp, atom_pad, rw_W, rw_b):
    grid = NP_ // 1024
    return pl.pallas_call(
        _enc_body,
        grid=(grid,),
        in_specs=[
            pl.BlockSpec((1024, 1), lambda i: (i, 0)),
            pl.BlockSpec((1024, 16), lambda i: (i, 0)),
            pl.BlockSpec((128, 128), lambda i: (0, 0)),
            pl.BlockSpec((16, 128), lambda i: (0, 0)),
            pl.BlockSpec((1, 128), lambda i: (0, 0)),
        ],
        out_specs=pl.BlockSpec((4, 1024, 32), lambda i: (0, i, 0)),
        out_shape=_sds((4, NP_, 32)),
    )(xnp, rwp, atom_pad, rw_W, rw_b)


def _mpb_body(node_ref, bond_ref, out_ref):
    a = pl.program_id(1)
    brow = bond_ref[:, pl.ds(a, 1), :]
    out_ref[...] = jnp.maximum(node_ref[...] + brow, 0.0)


def _mpb_tc(node4, bond4):
    nb = NP_ // 1024
    return pl.pallas_call(
        _mpb_body,
        grid=(nb, 10),
        in_specs=[
            pl.BlockSpec((4, 1024, 32), lambda i, a: (0, i, 0)),
            pl.BlockSpec((4, 16, 32), lambda i, a: (0, 0, 0)),
        ],
        out_specs=pl.BlockSpec((4, 1024, 32), lambda i, a: (0, a * nb + i, 0)),
        out_shape=_sds((4, TBL, 32)),
    )(node4, bond4)


def _gnn_body(x_ref, a_ref, w_ref, b_ref, out_ref):
    xc = jnp.concatenate([x_ref[q] for q in range(4)], axis=1)
    ac = jnp.concatenate([a_ref[q] for q in range(4)], axis=1)
    h = jnp.maximum(jnp.dot(xc + ac, w_ref[...],
                            preferred_element_type=_f32) + b_ref[...], 0.0)
    xn = xc + h
    out_ref[...] = jnp.concatenate(
        [xn[:, 32 * q:32 * (q + 1)].reshape(1, 256, 32) for q in range(4)],
        axis=0)


def _gnn_tc(x4, a4, W, b):
    grid = NSP // 256
    return pl.pallas_call(
        _gnn_body,
        grid=(grid,),
        in_specs=[
            pl.BlockSpec((4, 256, 32), lambda i: (0, i, 0)),
            pl.BlockSpec((4, 256, 32), lambda i: (0, i, 0)),
            pl.BlockSpec((128, 128), lambda i: (0, 0)),
            pl.BlockSpec((1, 128), lambda i: (0, 0)),
        ],
        out_specs=pl.BlockSpec((4, 256, 32), lambda i: (0, i, 0)),
        out_shape=_sds((4, NSP, 32)),
    )(x4, a4, W, b)


def _u_body(sb_ref, cnt_ref, w_ref, b_ref, out_ref):
    sc = jnp.concatenate([sb_ref[q] for q in range(4)], axis=1)
    pm = sc / jnp.maximum(cnt_ref[...][:, 0:1], 1.0)
    r = jnp.maximum(jnp.dot(pm, w_ref[...],
                            preferred_element_type=_f32) + b_ref[...], 0.0)
    out_ref[...] = jnp.concatenate(
        [r[:, 32 * q:32 * (q + 1)].reshape(1, SB, 32) for q in range(4)],
        axis=0)


def _u_tc(sb4, cntb, W, b):
    return pl.pallas_call(_u_body, out_shape=_sds((4, SB, 32)))(
        sb4, cntb, W, b)


def _ln(x):
    m = x.mean(-1, keepdims=True)
    v = x.var(-1, keepdims=True)
    return (x - m) * lax.rsqrt(v + 1e-5)


def _mix_body(sb_ref, cnt_ref, pe_ref, pw_ref, pb_ref, adj_ref, maskf_ref,
              wq_ref, wk_ref, wv_ref, wo_ref, w1_ref, b1_ref, w2_ref, b2_ref,
              ow1_ref, ob1_ref, ow2_ref, ob2_ref, out_ref):
    sc = jnp.concatenate([sb_ref[q] for q in range(4)], axis=1)
    mx = sc[:TP] / jnp.maximum(cnt_ref[...][:TP, 0:1], 1.0)
    mx = mx + jnp.dot(pe_ref[...][:TP], pw_ref[...],
                      preferred_element_type=_f32) + pb_ref[...]
    maskf = maskf_ref[...]
    scale = 1.0 / (H ** 0.5)
    for l in range(NL_MIX):
        h = _ln(mx)
        new_rows = []
        for b in range(B):
            hb = h[b * P:(b + 1) * P, :]
            qb = jnp.dot(hb, wq_ref[l], preferred_element_type=_f32)
            kb = jnp.dot(hb, wk_ref[l], preferred_element_type=_f32)
            vb = jnp.dot(hb, wv_ref[l], preferred_element_type=_f32)
            s = jnp.dot(qb, kb.T, preferred_element_type=_f32) * scale
            s = jnp.where((maskf[b] == 0.0)[None, :], -1e9, s)
            s = s - jnp.max(s, axis=-1, keepdims=True)
            e = jnp.exp(s)
            att = e / jnp.sum(e, axis=-1, keepdims=True)
            att = att * adj_ref[b]
            ob = jnp.dot(jnp.dot(att, vb, preferred_element_type=_f32),
                         wo_ref[l], preferred_element_type=_f32)
            new_rows.append(ob)
        mx = mx + jnp.concatenate(new_rows, axis=0)
        h2 = _ln(mx)
        ff = jnp.maximum(jnp.dot(h2, w1_ref[l],
                                 preferred_element_type=_f32) + b1_ref[l], 0.0)
        mx = mx + jnp.dot(ff, w2_ref[l],
                          preferred_element_type=_f32) + b2_ref[l]
    mx3 = mx.reshape(B, P, H)
    wsum = jnp.sum(mx3 * maskf[:, :, None], axis=1)
    cnt = jnp.sum(maskf, axis=1, keepdims=True)
    g = wsum / cnt
    dec = jnp.maximum(jnp.dot(g, ow1_ref[...],
                              preferred_element_type=_f32) + ob1_ref[...], 0.0)
    out_ref[...] = jnp.dot(dec, ow2_ref[...],
                           preferred_element_type=_f32) + ob2_ref[...]


def _mix_tc(sb4, cntb, pe, pW, pb, adj, maskf, Wq, Wk, Wv, Wo, ffW1, ffb1,
            ffW2, ffb2, oW1, ob1, oW2, ob2):
    return pl.pallas_call(_mix_body, out_shape=_sds((B, 1)))(
        sb4, cntb, pe, pW, pb, adj, maskf, Wq, Wk, Wv, Wo, ffW1, ffb1,
        ffW2, ffb2, oW1, ob1, oW2, ob2)


def kernel(x_node, rw_pos_enc, edge_attr, subgraphs_nodes_mapper,
           subgraphs_edges_mapper, combined_subgraphs, subgraphs_batch,
           patch_pe, coarsen_adj, mask, atom_emb, bond_emb, rw_W, rw_b,
           patch_W, patch_b, gnn_W, gnn_b, U_W, U_b, Wq, Wk, Wv, Wo,
           ffW1, ffb1, ffW2, ffb2, oW1, ob1, oW2, ob2):
    i32 = _i32
    snm = subgraphs_nodes_mapper.astype(i32)
    sem = subgraphs_edges_mapper.astype(i32)
    batch = subgraphs_batch.astype(i32)
    src = combined_subgraphs[0].astype(i32)
    dst = combined_subgraphs[1].astype(i32)

    snm_p = jnp.concatenate([snm, jnp.full((NSP - NS,), N, i32)])
    batch_p = jnp.concatenate([batch, jnp.full((NSP - NS,), TP, i32)])
    src_p = jnp.concatenate([src, jnp.zeros((ESP - ES,), i32)])
    dst_p = jnp.concatenate([dst, jnp.full((ESP - ES,), NS, i32)])
    sem_p = jnp.concatenate([sem, jnp.zeros((ESP - ES,), i32)])

    snm2 = snm_p.reshape(NT, RCH, 128)
    batch2 = batch_p.reshape(NT, RCH, 128)
    batchS2 = batch2 + NP_
    src2 = src_p.reshape(2 * NT, EWC, 128)
    dst2 = dst_p.reshape(ESP // 128, 128)
    sem2 = sem_p.reshape(2 * NT, EWC, 128)

    xn_p = jnp.concatenate([x_node.astype(i32),
                            jnp.zeros((NP_ - N,), i32)]).reshape(NP_, 1)
    rw_p = jnp.concatenate([rw_pos_enc,
                            jnp.zeros((NP_ - N, 16), _f32)], axis=0)
    atom_pad = jnp.concatenate([atom_emb,
                                jnp.zeros((28, H), _f32)], axis=0)
    bond4 = jnp.concatenate(
        [bond_emb.reshape(10, 4, 32).transpose(1, 0, 2),
         jnp.zeros((4, 6, 32), _f32)], axis=1)
    pe_p = jnp.concatenate([patch_pe, jnp.zeros((SB - TP, 8), _f32)], axis=0)
    maskf = mask.astype(_f32)
    zrows = jnp.zeros((NSP, 32), _f32)
    ones_in = jnp.ones((128, 32), _f32)

    enc4 = _enc_tc(xn_p, rw_p, atom_pad, rw_W, rw_b.reshape(1, H))
    enc_flat = enc4.reshape(4 * NP_, 32)
    eidx3, cntn, cntb, x_flat = _prep_sc(_prep_body)(
        sem2, src2, edge_attr.astype(i32), snm, batchS2, snm2, ones_in,
        zrows, enc_flat)
    eidx2 = eidx3.reshape(ESP // 128, 128)
    node4 = enc4
    for i in range(NL_GNN):
        if i > 0:
            sb = _bsum_sc(_bsum_body)(x_flat, batch2, zrows)
            R4 = _u_tc(sb.reshape(4, SB, 32), cntb, U_W[i - 1],
                       U_b[i - 1].reshape(1, H))
            node_flat, x_flat = _node_sc(_node_body)(
                x_flat, R4.reshape(4 * SB, 32), batch2, snm2, cntn, zrows)
            node4 = node_flat.reshape(4, NP_, 32)
        tbl4 = _mpb_tc(node4, bond4)
        agg_flat = _edge_sc(_edge_body)(tbl4.reshape(4 * TBL, 32), eidx2,
                                        dst2, zrows)
        x4 = _gnn_tc(x_flat.reshape(4, NSP, 32), agg_flat.reshape(4, NSP, 32),
                     gnn_W[i], gnn_b[i].reshape(1, H))
        x_flat = x4.reshape(4 * NSP, 32)

    sb = _bsum_sc(_bsum_body)(x_flat, batch2, zrows)
    return _mix_tc(sb.reshape(4, SB, 32), cntb, pe_p, patch_W,
                   patch_b.reshape(1, H), coarsen_adj, maskf, Wq, Wk, Wv, Wo,
                   ffW1, ffb1, ffW2, ffb2, oW1, ob1, oW2, ob2)


# speedup vs baseline: 2.1062x; 1.1175x over previous
import functools

import jax
import jax.numpy as jnp
from jax import lax
from jax.experimental import pallas as pl
from jax.experimental.pallas import tpu as pltpu
from jax.experimental.pallas import tpu_sc as plsc

N = 10000; E = 320000; NS = 40000; ES = 400000
B = 8; P = 32; TP = B * P; H = 128
NL_GNN = 4; NL_MIX = 2

NP_ = 10240
NSP = 40960
ESP = 409600
SB = 272
TBL = 10 * NP_
NT = 16
EPT = ESP // NT
ECH = EPT // 128
RPT = NSP // NT
RCH = RPT // 128
NPT = NP_ // NT
EW = ESP // (2 * NT)
EWC = EW // 128

_f32 = jnp.float32
_i32 = jnp.int32


def _sds(shape, dtype=_f32):
    return jax.ShapeDtypeStruct(shape, dtype)


def _al(x):
    return pl.multiple_of(x, 8)


def _prep_body(sem2, src2, ea_t, snm_t, batchS2, snm2, ones_in, zrows,
               enc_all, eidx_out, cntn_out, cntb_out, x0_out,
               cS, b1, b2, jb, kb, av, nv, rows, ov, gs0, gs1):
    c = lax.axis_index("c")
    s = lax.axis_index("s")
    w = c * NT + s
    pltpu.sync_copy(sem2.at[w], b1)
    pltpu.sync_copy(src2.at[w], b2)

    def idx_chunk(k, _):
        pltpu.async_copy(ea_t.at[b1.at[k]], av, gs0).wait()
        pltpu.async_copy(snm_t.at[b2.at[k]], nv, gs1).wait()
        for i in range(8):
            sl = pl.ds(i * 16, 16)
            b1[k, sl] = av[sl] * NP_ + nv[sl]
        return 0

    lax.fori_loop(0, EWC, idx_chunk, 0)
    pltpu.sync_copy(b1, eidx_out.at[w])

    pltpu.sync_copy(snm2.at[s], jb)
    pltpu.sync_copy(batchS2.at[s], kb)
    pltpu.sync_copy(ones_in, ov)
    ZPT = (NP_ + 512) // NT
    pltpu.sync_copy(zrows.at[pl.ds(_al(s * ZPT), ZPT)],
                    cS.at[pl.ds(_al(s * ZPT), ZPT)])
    plsc.subcore_barrier()

    def cnt_chunk(k, _):
        pltpu.sync_copy(ov, cS.at[jb.at[k]], add=True)
        pltpu.sync_copy(ov, cS.at[kb.at[k]], add=True)
        return 0

    lax.fori_loop(0, RCH, cnt_chunk, 0)
    plsc.subcore_barrier()

    @pl.when(c == 0)
    def _():
        pltpu.sync_copy(cS.at[pl.ds(_al(s * NPT), NPT)],
                        cntn_out.at[pl.ds(_al(s * NPT), NPT)])

    @pl.when(jnp.logical_and(c == 1, s == 0))
    def _():
        pltpu.sync_copy(cS.at[pl.ds(NP_, SB)], cntb_out)

    for j in range(2):
        q = c * 2 + j

        def g_chunk(k, _):
            base = s * RPT + k * 128
            for i in range(8):
                sl = pl.ds(i * 16, 16)
                kb[k, sl] = jb[k, sl] + q * NP_
            pltpu.async_copy(enc_all.at[kb.at[k]], rows, gs0).wait()
            pltpu.sync_copy(rows, x0_out.at[pl.ds(_al(q * NSP + base), 128)])
            return 0

        lax.fori_loop(0, RCH, g_chunk, 0)


def _edge_body(tb_all, eidx2, dst2, zrows, agg_out,
               aggS, eb, db, eb2, rows, g0, g1, g2, g3, s0, s1, s2, s3):
    c = lax.axis_index("c")
    s = lax.axis_index("s")
    gsems = (g0, g1, g2, g3)
    ssems = (s0, s1, s2, s3)
    CPB = 40
    NBLK = ECH // CPB
    for j in range(2):
        q = c * 2 + j
        pltpu.sync_copy(zrows.at[pl.ds(_al(s * RPT), RPT)],
                        aggS.at[pl.ds(_al(s * RPT), RPT)])
        plsc.subcore_barrier()

        def block(bk, _):
            base = _al(s * ECH + bk * CPB)
            pltpu.sync_copy(eidx2.at[pl.ds(base, CPB)], eb)
            pltpu.sync_copy(dst2.at[pl.ds(base, CPB)], db)

            def shift_chunk(k, _):
                for i in range(8):
                    sl = pl.ds(i * 16, 16)
                    eb2[k, sl] = eb[k, sl] + q * TBL
                return 0

            lax.fori_loop(0, CPB, shift_chunk, 0)

            def quad(m, _):
                for b in range(4):
                    k = 4 * m + b

                    @pl.when(k >= 4)
                    def _():
                        pltpu.make_async_copy(rows.at[b], aggS.at[db.at[k]],
                                              ssems[b]).wait()

                    pltpu.async_copy(tb_all.at[eb2.at[k]], rows.at[b],
                                     gsems[b])
                for b in range(4):
                    k = 4 * m + b
                    pltpu.make_async_copy(tb_all.at[eb2.at[k]], rows.at[b],
                                          gsems[b]).wait()
                    pltpu.async_copy(rows.at[b], aggS.at[db.at[k]], ssems[b],
                                     add=True)
                return 0

            lax.fori_loop(0, CPB // 4, quad, 0)
            for b in range(4):
                pltpu.make_async_copy(rows.at[b], aggS.at[db.at[36 + b]],
                                      ssems[b]).wait()
            return 0

        lax.fori_loop(0, NBLK, block, 0)
        plsc.subcore_barrier()
        pltpu.sync_copy(aggS.at[pl.ds(_al(s * RPT), RPT)],
                        agg_out.at[pl.ds(_al(q * NSP + s * RPT), RPT)])
        plsc.subcore_barrier()


def _bsum_body(x_all, batch2, zrows, sb_out, sS, ib, rows, gs0, gs1):
    c = lax.axis_index("c")
    s = lax.axis_index("s")
    pltpu.sync_copy(batch2.at[s], ib)
    for j in range(2):
        q = c * 2 + j

        @pl.when(s == 0)
        def _():
            pltpu.sync_copy(zrows.at[pl.ds(0, SB)], sS)

        plsc.subcore_barrier()

        pltpu.async_copy(x_all.at[pl.ds(_al(q * NSP + s * RPT), 128)],
                         rows.at[0], gs0)

        def pair(p, _):
            base = q * NSP + s * RPT
            c0 = 2 * p
            pltpu.make_async_copy(
                x_all.at[pl.ds(_al(base + c0 * 128), 128)], rows.at[0],
                gs0).wait()
            pltpu.async_copy(x_all.at[pl.ds(_al(base + (c0 + 1) * 128), 128)],
                             rows.at[1], gs1)
            pltpu.sync_copy(rows.at[0], sS.at[ib.at[c0]], add=True)
            pltpu.make_async_copy(
                x_all.at[pl.ds(_al(base + (c0 + 1) * 128), 128)], rows.at[1],
                gs1).wait()

            @pl.when(p < RCH // 2 - 1)
            def _():
                pltpu.async_copy(
                    x_all.at[pl.ds(_al(base + (c0 + 2) * 128), 128)],
                    rows.at[0], gs0)

            pltpu.sync_copy(rows.at[1], sS.at[ib.at[c0 + 1]], add=True)
            return 0

        lax.fori_loop(0, RCH // 2, pair, 0)
        plsc.subcore_barrier()
        pltpu.sync_copy(sS.at[pl.ds(_al(s * 16), 16)],
                        sb_out.at[pl.ds(_al(q * SB + s * 16), 16)])

        @pl.when(s == 0)
        def _():
            pltpu.sync_copy(sS.at[pl.ds(256, 16)],
                            sb_out.at[pl.ds(_al(q * SB + 256), 16)])

        plsc.subcore_barrier()


def _node_body(x_all, R_all, batch2, snm2, cntn, zrows, node_out, xn_out,
               nS, ib, jb, jb2, rows, rrows, sbuf, tbuf, gs0, gs1):
    c = lax.axis_index("c")
    s = lax.axis_index("s")
    pltpu.sync_copy(batch2.at[s], ib)
    pltpu.sync_copy(snm2.at[s], jb)
    for j in range(2):
        q = c * 2 + j
        pltpu.sync_copy(zrows.at[pl.ds(_al(s * NPT), NPT)],
                        nS.at[pl.ds(_al(s * NPT), NPT)])

        def shift_chunk(k, _):
            for i in range(8):
                sl = pl.ds(i * 16, 16)
                jb2[k, sl] = ib[k, sl] + q * SB
            return 0

        lax.fori_loop(0, RCH, shift_chunk, 0)
        plsc.subcore_barrier()

        def acc_chunk(k, _):
            base = _al(q * NSP + s * RPT + k * 128)
            pltpu.async_copy(x_all.at[pl.ds(base, 128)], rows, gs0)
            pltpu.async_copy(R_all.at[jb2.at[k]], rrows, gs1)
            pltpu.make_async_copy(x_all.at[pl.ds(base, 128)], rows, gs0).wait()
            pltpu.make_async_copy(R_all.at[jb2.at[k]], rrows, gs1).wait()

            def addrow(r, _):
                for hh in range(2):
                    sl = pl.ds(hh * 16, 16)
                    rows[r, sl] = rows[r, sl] + rrows[r, sl]
                return 0

            lax.fori_loop(0, 128, addrow, 0)
            pltpu.sync_copy(rows, nS.at[jb.at[k]], add=True)
            return 0

        lax.fori_loop(0, RCH, acc_chunk, 0)
        plsc.subcore_barrier()
        pltpu.sync_copy(nS.at[pl.ds(_al(s * NPT), NPT)], sbuf)
        pltpu.sync_copy(cntn.at[pl.ds(_al(s * NPT), NPT)], tbuf)

        def scale_row(r, _):
            for hh in range(2):
                sl = pl.ds(hh * 16, 16)
                sbuf[r, sl] = sbuf[r, sl] / jnp.maximum(tbuf[r, sl], 1.0)
            return 0

        lax.fori_loop(0, NPT, scale_row, 0)
        pltpu.sync_copy(sbuf,
                        node_out.at[pl.ds(_al(q * NP_ + s * NPT), NPT)])
        plsc.subcore_barrier()

        def gb_chunk(k, _):
            for i in range(8):
                sl = pl.ds(i * 16, 16)
                jb2[k, sl] = jb[k, sl] + q * NP_
            base = _al(q * NSP + s * RPT + k * 128)
            pltpu.async_copy(node_out.at[jb2.at[k]], rows, gs0).wait()
            pltpu.sync_copy(rows, xn_out.at[pl.ds(base, 128)])
            return 0

        lax.fori_loop(0, RCH, gb_chunk, 0)


def _mesh():
    return plsc.VectorSubcoreMesh(core_axis_name="c", subcore_axis_name="s",
                                  num_cores=2, num_subcores=NT)


def _prep_sc(body):
  return pl.kernel(
    body,
    compiler_params=pltpu.CompilerParams(use_tc_tiling_on_sc=False),
    out_type=(_sds((2 * NT, EWC, 128), _i32), _sds((NP_, 32)),
              _sds((SB, 32)), _sds((4 * NSP, 32))),
    mesh=_mesh(),
    scratch_types=[
        pltpu.VMEM_SHARED((NP_ + 512, 32), _f32),
        pltpu.VMEM((EWC, 128), _i32),
        pltpu.VMEM((EWC, 128), _i32),
        pltpu.VMEM((RCH, 128), _i32),
        pltpu.VMEM((RCH, 128), _i32),
        pltpu.VMEM((128,), _i32),
        pltpu.VMEM((128,), _i32),
        pltpu.VMEM((128, 32), _f32),
        pltpu.VMEM((128, 32), _f32),
        pltpu.SemaphoreType.DMA,
        pltpu.SemaphoreType.DMA,
    ])

def _edge_sc(body):
  return pl.kernel(
    body,
    compiler_params=pltpu.CompilerParams(use_tc_tiling_on_sc=False),
    out_type=_sds((4 * NSP, 32)),
    mesh=_mesh(),
    scratch_types=[
        pltpu.VMEM_SHARED((NSP, 32), _f32),
        pltpu.VMEM((40, 128), _i32),
        pltpu.VMEM((40, 128), _i32),
        pltpu.VMEM((40, 128), _i32),
        pltpu.VMEM((4, 128, 32), _f32),
        pltpu.SemaphoreType.DMA,
        pltpu.SemaphoreType.DMA,
        pltpu.SemaphoreType.DMA,
        pltpu.SemaphoreType.DMA,
        pltpu.SemaphoreType.DMA,
        pltpu.SemaphoreType.DMA,
        pltpu.SemaphoreType.DMA,
        pltpu.SemaphoreType.DMA,
    ])

def _bsum_sc(body):
  return pl.kernel(
    body,
    compiler_params=pltpu.CompilerParams(use_tc_tiling_on_sc=False),
    out_type=_sds((4 * SB, 32)),
    mesh=_mesh(),
    scratch_types=[
        pltpu.VMEM_SHARED((SB, 32), _f32),
        pltpu.VMEM((RCH, 128), _i32),
        pltpu.VMEM((2, 128, 32), _f32),
        pltpu.SemaphoreType.DMA,
        pltpu.SemaphoreType.DMA,
    ])

def _node_sc(body):
  return pl.kernel(
    body,
    compiler_params=pltpu.CompilerParams(use_tc_tiling_on_sc=False),
    out_type=(_sds((4 * NP_, 32)), _sds((4 * NSP, 32))),
    mesh=_mesh(),
    scratch_types=[
        pltpu.VMEM_SHARED((NP_, 32), _f32),
        pltpu.VMEM((RCH, 128), _i32),
        pltpu.VMEM((RCH, 128), _i32),
        pltpu.VMEM((RCH, 128), _i32),
        pltpu.VMEM((128, 32), _f32),
        pltpu.VMEM((128, 32), _f32),
        pltpu.VMEM((NPT, 32), _f32),
        pltpu.VMEM((NPT, 32), _f32),
        pltpu.SemaphoreType.DMA,
        pltpu.SemaphoreType.DMA,
    ])


def _enc_body(xn_ref, rw_ref, atom_ref, rww_ref, rwb_ref, out_ref):
    xn = xn_ref[...]
    oh = (xn == lax.broadcasted_iota(_i32, (1024, 128), 1)).astype(_f32)
    enc = jnp.dot(oh, atom_ref[...], preferred_element_type=_f32)
    enc = enc + jnp.dot(rw_ref[...], rww_ref[...],
                        preferred_element_type=_f32) + rwb_ref[...]
    out_ref[...] = jnp.concatenate(
        [enc[:, 32 * q:32 * (q + 1)].reshape(1, 1024, 32) for q in range(4)],
        axis=0)


def _enc_tc(xnp, rwp, atom_pad, rw_W, rw_b):
    grid = NP_ // 1024
    return pl.pallas_call(
        _enc_body,
        grid=(grid,),
        in_specs=[
            pl.BlockSpec((1024, 1), lambda i: (i, 0)),
            pl.BlockSpec((1024, 16), lambda i: (i, 0)),
            pl.BlockSpec((128, 128), lambda i: (0, 0)),
            pl.BlockSpec((16, 128), lambda i: (0, 0)),
            pl.BlockSpec((1, 128), lambda i: (0, 0)),
        ],
        out_specs=pl.BlockSpec((4, 1024, 32), lambda i: (0, i, 0)),
        out_shape=_sds((4, NP_, 32)),
    )(xnp, rwp, atom_pad, rw_W, rw_b)


def _mpb_body(node_ref, bond_ref, out_ref):
    a = pl.program_id(1)
    brow = bond_ref[:, pl.ds(a, 1), :]
    out_ref[...] = jnp.maximum(node_ref[...] + brow, 0.0)


def _mpb_tc(node4, bond4):
    nb = NP_ // 1024
    return pl.pallas_call(
        _mpb_body,
        grid=(nb, 10),
        in_specs=[
            pl.BlockSpec((4, 1024, 32), lambda i, a: (0, i, 0)),
            pl.BlockSpec((4, 16, 32), lambda i, a: (0, 0, 0)),
        ],
        out_specs=pl.BlockSpec((4, 1024, 32), lambda i, a: (0, a * nb + i, 0)),
        out_shape=_sds((4, TBL, 32)),
    )(node4, bond4)


def _gnn_body(x_ref, a_ref, w_ref, b_ref, out_ref):
    xc = jnp.concatenate([x_ref[q] for q in range(4)], axis=1)
    ac = jnp.concatenate([a_ref[q] for q in range(4)], axis=1)
    h = jnp.maximum(jnp.dot(xc + ac, w_ref[...],
                            preferred_element_type=_f32) + b_ref[...], 0.0)
    xn = xc + h
    out_ref[...] = jnp.concatenate(
        [xn[:, 32 * q:32 * (q + 1)].reshape(1, 256, 32) for q in range(4)],
        axis=0)


def _gnn_tc(x4, a4, W, b):
    grid = NSP // 256
    return pl.pallas_call(
        _gnn_body,
        grid=(grid,),
        in_specs=[
            pl.BlockSpec((4, 256, 32), lambda i: (0, i, 0)),
            pl.BlockSpec((4, 256, 32), lambda i: (0, i, 0)),
            pl.BlockSpec((128, 128), lambda i: (0, 0)),
            pl.BlockSpec((1, 128), lambda i: (0, 0)),
        ],
        out_specs=pl.BlockSpec((4, 256, 32), lambda i: (0, i, 0)),
        out_shape=_sds((4, NSP, 32)),
    )(x4, a4, W, b)


def _u_body(sb_ref, cnt_ref, w_ref, b_ref, out_ref):
    sc = jnp.concatenate([sb_ref[q] for q in range(4)], axis=1)
    pm = sc / jnp.maximum(cnt_ref[...][:, 0:1], 1.0)
    r = jnp.maximum(jnp.dot(pm, w_ref[...],
                            preferred_element_type=_f32) + b_ref[...], 0.0)
    out_ref[...] = jnp.concatenate(
        [r[:, 32 * q:32 * (q + 1)].reshape(1, SB, 32) for q in range(4)],
        axis=0)


def _u_tc(sb4, cntb, W, b):
    return pl.pallas_call(_u_body, out_shape=_sds((4, SB, 32)))(
        sb4, cntb, W, b)


def _ln(x):
    m = x.mean(-1, keepdims=True)
    v = x.var(-1, keepdims=True)
    return (x - m) * lax.rsqrt(v + 1e-5)


def _mix_body(sb_ref, cnt_ref, pe_ref, pw_ref, pb_ref, adj_ref, maskf_ref,
              wq_ref, wk_ref, wv_ref, wo_ref, w1_ref, b1_ref, w2_ref, b2_ref,
              ow1_ref, ob1_ref, ow2_ref, ob2_ref, out_ref):
    sc = jnp.concatenate([sb_ref[q] for q in range(4)], axis=1)
    mx = sc[:TP] / jnp.maximum(cnt_ref[...][:TP, 0:1], 1.0)
    mx = mx + jnp.dot(pe_ref[...][:TP], pw_ref[...],
                      preferred_element_type=_f32) + pb_ref[...]
    maskf = maskf_ref[...]
    scale = 1.0 / (H ** 0.5)
    for l in range(NL_MIX):
        h = _ln(mx)
        new_rows = []
        for b in range(B):
            hb = h[b * P:(b + 1) * P, :]
            qb = jnp.dot(hb, wq_ref[l], preferred_element_type=_f32)
            kb = jnp.dot(hb, wk_ref[l], preferred_element_type=_f32)
            vb = jnp.dot(hb, wv_ref[l], preferred_element_type=_f32)
            s = jnp.dot(qb, kb.T, preferred_element_type=_f32) * scale
            s = jnp.where((maskf[b] == 0.0)[None, :], -1e9, s)
            s = s - jnp.max(s, axis=-1, keepdims=True)
            e = jnp.exp(s)
            att = e / jnp.sum(e, axis=-1, keepdims=True)
            att = att * adj_ref[b]
            ob = jnp.dot(jnp.dot(att, vb, preferred_element_type=_f32),
                         wo_ref[l], preferred_element_type=_f32)
            new_rows.append(ob)
        mx = mx + jnp.concatenate(new_rows, axis=0)
        h2 = _ln(mx)
        ff = jnp.maximum(jnp.dot(h2, w1_ref[l],
                                 preferred_element_type=_f32) + b1_ref[l], 0.0)
        mx = mx + jnp.dot(ff, w2_ref[l],
                          preferred_element_type=_f32) + b2_ref[l]
    mx3 = mx.reshape(B, P, H)
    wsum = jnp.sum(mx3 * maskf[:, :, None], axis=1)
    cnt = jnp.sum(maskf, axis=1, keepdims=True)
    g = wsum / cnt
    dec = jnp.maximum(jnp.dot(g, ow1_ref[...],
                              preferred_element_type=_f32) + ob1_ref[...], 0.0)
    out_ref[...] = jnp.dot(dec, ow2_ref[...],
                           preferred_element_type=_f32) + ob2_ref[...]


def _mix_tc(sb4, cntb, pe, pW, pb, adj, maskf, Wq, Wk, Wv, Wo, ffW1, ffb1,
            ffW2, ffb2, oW1, ob1, oW2, ob2):
    return pl.pallas_call(_mix_body, out_shape=_sds((B, 1)))(
        sb4, cntb, pe, pW, pb, adj, maskf, Wq, Wk, Wv, Wo, ffW1, ffb1,
        ffW2, ffb2, oW1, ob1, oW2, ob2)


def kernel(x_node, rw_pos_enc, edge_attr, subgraphs_nodes_mapper,
           subgraphs_edges_mapper, combined_subgraphs, subgraphs_batch,
           patch_pe, coarsen_adj, mask, atom_emb, bond_emb, rw_W, rw_b,
           patch_W, patch_b, gnn_W, gnn_b, U_W, U_b, Wq, Wk, Wv, Wo,
           ffW1, ffb1, ffW2, ffb2, oW1, ob1, oW2, ob2):
    i32 = _i32
    snm = subgraphs_nodes_mapper.astype(i32)
    sem = subgraphs_edges_mapper.astype(i32)
    batch = subgraphs_batch.astype(i32)
    src = combined_subgraphs[0].astype(i32)
    dst = combined_subgraphs[1].astype(i32)

    snm_p = jnp.concatenate([snm, jnp.full((NSP - NS,), N, i32)])
    batch_p = jnp.concatenate([batch, jnp.full((NSP - NS,), TP, i32)])
    src_p = jnp.concatenate([src, jnp.zeros((ESP - ES,), i32)])
    dst_p = jnp.concatenate([dst, jnp.full((ESP - ES,), NS, i32)])
    sem_p = jnp.concatenate([sem, jnp.zeros((ESP - ES,), i32)])

    snm2 = snm_p.reshape(NT, RCH, 128)
    batch2 = batch_p.reshape(NT, RCH, 128)
    batchS2 = batch2 + NP_
    src2 = src_p.reshape(2 * NT, EWC, 128)
    dst2 = dst_p.reshape(ESP // 128, 128)
    sem2 = sem_p.reshape(2 * NT, EWC, 128)

    xn_p = jnp.concatenate([x_node.astype(i32),
                            jnp.zeros((NP_ - N,), i32)]).reshape(NP_, 1)
    rw_p = jnp.concatenate([rw_pos_enc,
                            jnp.zeros((NP_ - N, 16), _f32)], axis=0)
    atom_pad = jnp.concatenate([atom_emb,
                                jnp.zeros((28, H), _f32)], axis=0)
    bond4 = jnp.concatenate(
        [bond_emb.reshape(10, 4, 32).transpose(1, 0, 2),
         jnp.zeros((4, 6, 32), _f32)], axis=1)
    pe_p = jnp.concatenate([patch_pe, jnp.zeros((SB - TP, 8), _f32)], axis=0)
    maskf = mask.astype(_f32)
    zrows = jnp.zeros((NSP, 32), _f32)
    ones_in = jnp.ones((128, 32), _f32)

    enc4 = _enc_tc(xn_p, rw_p, atom_pad, rw_W, rw_b.reshape(1, H))
    enc_flat = enc4.reshape(4 * NP_, 32)
    eidx3, cntn, cntb, x_flat = _prep_sc(_prep_body)(
        sem2, src2, edge_attr.astype(i32), snm, batchS2, snm2, ones_in,
        zrows, enc_flat)
    eidx2 = eidx3.reshape(ESP // 128, 128)
    node4 = enc4
    for i in range(NL_GNN):
        if i > 0:
            sb = _bsum_sc(_bsum_body)(x_flat, batch2, zrows)
            R4 = _u_tc(sb.reshape(4, SB, 32), cntb, U_W[i - 1],
                       U_b[i - 1].reshape(1, H))
            node_flat, x_flat = _node_sc(_node_body)(
                x_flat, R4.reshape(4 * SB, 32), batch2, snm2, cntn, zrows)
            node4 = node_flat.reshape(4, NP_, 32)
        tbl4 = _mpb_tc(node4, bond4)
        agg_flat = _edge_sc(_edge_body)(tbl4.reshape(4 * TBL, 32), eidx2,
                                        dst2, zrows)
        x4 = _gnn_tc(x_flat.reshape(4, NSP, 32), agg_flat.reshape(4, NSP, 32),
                     gnn_W[i], gnn_b[i].reshape(1, H))
        x_flat = x4.reshape(4 * NSP, 32)

    sb = _bsum_sc(_bsum_body)(x_flat, batch2, zrows)
    return _mix_tc(sb.reshape(4, SB, 32), cntb, pe_p, patch_W,
                   patch_b.reshape(1, H), coarsen_adj, maskf, Wq, Wk, Wv, Wo,
                   ffW1, ffb1, ffW2, ffb2, oW1, ob1, oW2, ob2)
